# baseline (speedup 1.0000x reference)
import functools

import jax
import jax.numpy as jnp
from jax import lax
from jax.experimental import pallas as pl
from jax.experimental.pallas import tpu as pltpu
from jax.experimental.pallas import tpu_sc as plsc

N_W = 2097152
K = 1024
W = 16
NC = 2
NS = 16
L = 16
NW = NC * NS
PER_W = N_W // NW
CH = 32768
N_SUB = PER_W // CH
BIG = jnp.int32(1 << 30)


def _tree_min(vals):
    vals = list(vals)
    while len(vals) > 1:
        nxt = [jnp.minimum(vals[i], vals[i + 1]) for i in range(0, len(vals) - 1, 2)]
        if len(vals) % 2:
            nxt.append(vals[-1])
        vals = nxt
    return vals[0]


def _mesh():
    return plsc.VectorSubcoreMesh(
        core_axis_name="c", subcore_axis_name="s", num_cores=NC, num_subcores=NS
    )


@functools.partial(
    pl.kernel,
    out_type=jax.ShapeDtypeStruct((N_W,), jnp.int32),
    mesh=_mesh(),
    scratch_types=[
        pltpu.VMEM((K,), jnp.float32),
        pltpu.VMEM((K,), jnp.float32),
        pltpu.VMEM((K,), jnp.int32),
        pltpu.VMEM((CH,), jnp.float32),
        pltpu.VMEM((CH,), jnp.int32),
    ],
    compiler_params=pltpu.CompilerParams(
        needs_layout_passes=False, disable_bounds_checks=True
    ),
)
def _vq_kernel(w_hbm, csort_hbm, order_hbm, out_hbm,
               csort_v, csq_v, order_v, xbuf, obuf):
    wid = lax.axis_index("s") * NC + lax.axis_index("c")
    pltpu.sync_copy(csort_hbm, csort_v)
    pltpu.sync_copy(order_hbm, order_v)

    def csq_body(i, _):
        c = csort_v[pl.ds(i * L, L)]
        csq_v[pl.ds(i * L, L)] = c * c
        return 0

    lax.fori_loop(0, K // L, csq_body, 0)

    UNROLL = 4

    def run_chunk():
        @plsc.parallel_loop(0, CH // L, unroll=UNROLL)
        def group_body(g):
            x = xbuf[pl.ds(g * L, L)]
            xs = x * x
            lo = jnp.zeros((L,), jnp.int32)
            for half in (512, 256, 128, 64, 32, 16, 8, 4, 2, 1):
                cv = plsc.load_gather(csort_v, [lo + (half - 1)])
                lo = jnp.where(cv <= x, lo + half, lo)
            start = jnp.clip(lo - (W // 2 - 1), 0, K - W)

            bd = jnp.full((L,), jnp.inf, jnp.float32)
            bo = jnp.full((L,), BIG, jnp.int32)
            for w in range(W):
                cidx = start + w
                c = plsc.load_gather(csort_v, [cidx])
                csq = plsc.load_gather(csq_v, [cidx])
                og = plsc.load_gather(order_v, [cidx])
                t = x * c
                u = xs - 2.0 * t
                d = u + csq
                take = (d < bd) | ((d == bd) & (og < bo))
                bd = jnp.where(take, d, bd)
                bo = jnp.where(take, og, bo)
            obuf[pl.ds(g * L, L)] = bo

    for sub in range(N_SUB):
        base = wid * PER_W + sub * CH
        pltpu.sync_copy(w_hbm.at[pl.ds(base, CH)], xbuf)
        run_chunk()
        pltpu.sync_copy(obuf, out_hbm.at[pl.ds(base, CH)])


def kernel(weights_dict, y, codebook):
    c = codebook[:, 0]
    order = jnp.argsort(c).astype(jnp.int32)
    csort = c[order]
    indices = _vq_kernel(weights_dict, csort, order)
    return indices, y


# speedup vs baseline: 1.2137x; 1.2137x over previous
import functools

import jax
import jax.numpy as jnp
from jax import lax
from jax.experimental import pallas as pl
from jax.experimental.pallas import tpu as pltpu
from jax.experimental.pallas import tpu_sc as plsc

N_W = 2097152
K = 1024
W = 12
NC = 2
NS = 16
L = 16
NW = NC * NS
PER_W = N_W // NW
CH = 32768
N_SUB = PER_W // CH
BIG = jnp.int32(1 << 30)


def _tree_min(vals):
    vals = list(vals)
    while len(vals) > 1:
        nxt = [jnp.minimum(vals[i], vals[i + 1]) for i in range(0, len(vals) - 1, 2)]
        if len(vals) % 2:
            nxt.append(vals[-1])
        vals = nxt
    return vals[0]


def _mesh():
    return plsc.VectorSubcoreMesh(
        core_axis_name="c", subcore_axis_name="s", num_cores=NC, num_subcores=NS
    )


@functools.partial(
    pl.kernel,
    out_type=jax.ShapeDtypeStruct((N_W,), jnp.int32),
    mesh=_mesh(),
    scratch_types=[
        pltpu.VMEM((K,), jnp.float32),
        pltpu.VMEM((K,), jnp.int32),
        pltpu.VMEM((CH,), jnp.float32),
        pltpu.VMEM((CH,), jnp.int32),
    ],
    compiler_params=pltpu.CompilerParams(
        needs_layout_passes=False, disable_bounds_checks=True
    ),
)
def _vq_kernel(w_hbm, csort_hbm, order_hbm, out_hbm,
               csort_v, order_v, xbuf, obuf):
    wid = lax.axis_index("s") * NC + lax.axis_index("c")
    pltpu.sync_copy(csort_hbm, csort_v)
    pltpu.sync_copy(order_hbm, order_v)

    U = 8

    def group_body(gg, _):
        xs_u, lo_u = [], []
        for u in range(U):
            x = xbuf[pl.ds((gg * U + u) * L, L)]
            xs_u.append((x, x * x))
        for u in range(U):
            lo_u.append(jnp.zeros((L,), jnp.int32))
        for half in (512, 256, 128, 64, 32, 16, 8, 4, 2, 1):
            for u in range(U):
                cv = plsc.load_gather(csort_v, [lo_u[u] + (half - 1)])
                lo_u[u] = jnp.where(cv <= xs_u[u][0], lo_u[u] + half, lo_u[u])

        st_u = [jnp.clip(lo_u[u] - (W // 2 - 1), 0, K - W) for u in range(U)]
        bd_u = [jnp.full((L,), jnp.inf, jnp.float32) for _ in range(U)]
        bo_u = [jnp.full((L,), BIG, jnp.int32) for _ in range(U)]
        for w in range(W):
            for u in range(U):
                x, xs = xs_u[u]
                cidx = st_u[u] + w
                c = plsc.load_gather(csort_v, [cidx])
                og = plsc.load_gather(order_v, [cidx])
                t = x * c
                u_ = xs - 2.0 * t
                d = u_ + c * c
                take = (d < bd_u[u]) | ((d == bd_u[u]) & (og < bo_u[u]))
                bd_u[u] = jnp.where(take, d, bd_u[u])
                bo_u[u] = jnp.where(take, og, bo_u[u])
        for u in range(U):
            obuf[pl.ds((gg * U + u) * L, L)] = bo_u[u]
        return 0

    def run_chunk():
        lax.fori_loop(0, CH // (L * U), group_body, 0)

    for sub in range(N_SUB):
        base = wid * PER_W + sub * CH
        pltpu.sync_copy(w_hbm.at[pl.ds(base, CH)], xbuf)
        run_chunk()
        pltpu.sync_copy(obuf, out_hbm.at[pl.ds(base, CH)])


def kernel(weights_dict, y, codebook):
    c = codebook[:, 0]
    order = jnp.argsort(c).astype(jnp.int32)
    csort = c[order]
    indices = _vq_kernel(weights_dict, csort, order)
    return indices, y
